# baseline (speedup 1.0000x reference)
import functools

import jax
import jax.numpy as jnp
from jax import lax
from jax.experimental import pallas as pl
from jax.experimental.pallas import tpu as pltpu
from jax.experimental.pallas import tpu_sc as plsc

_HI = lax.Precision.DEFAULT

_NC = 2
_NS = 16
_NW = _NC * _NS
_LANES = 16


def _prep_body(batch_ref, posT_ref, t_ref, W_ta_ref, b_ta_ref, W_tb_ref,
               b_tb_ref, b_atom_ref, W_atom_ref, W_at_ref, b_at_ref,
               b_bond_ref, W_bond_ref, W_bt_ref, b_bt_ref,
               posc_ref, Wn_ref, We_ref):
    n = batch_ref.shape[1]
    g = t_ref.shape[0]
    ohT = (batch_ref[...] == lax.broadcasted_iota(jnp.int32, (g, n), 0)
           ).astype(jnp.float32)
    posT4 = jnp.concatenate(
        [posT_ref[...], jnp.ones((1, n), jnp.float32)], axis=0)
    sums = lax.dot_general(posT4, ohT, (((1,), (1,)), ((), ())),
                           precision=_HI, preferred_element_type=jnp.float32)
    mean = sums / jnp.maximum(sums[3:4, :], 1.0)
    poscT = posT4 - lax.dot_general(mean, ohT, (((1,), (0,)), ((), ())),
                                    precision=_HI,
                                    preferred_element_type=jnp.float32)
    posc_ref[pl.ds(0, n)] = poscT[0, :]
    posc_ref[pl.ds(n, n)] = poscT[1, :]
    posc_ref[pl.ds(2 * n, n)] = poscT[2, :]
    ta = t_ref[...] * W_ta_ref[...] + b_ta_ref[...]
    cnode = jnp.dot(ta + b_atom_ref[...], W_at_ref[...],
                    precision=_HI,
                    preferred_element_type=jnp.float32) + b_at_ref[...]
    tb = t_ref[...] * W_tb_ref[...] + b_tb_ref[...]
    cedge = jnp.dot(tb + b_bond_ref[...], W_bt_ref[...],
                    precision=_HI,
                    preferred_element_type=jnp.float32) + b_bt_ref[...]
    Wc = jnp.dot(W_atom_ref[...], W_at_ref[...], precision=_HI,
                 preferred_element_type=jnp.float32)
    We = jnp.dot(W_bond_ref[...], W_bt_ref[...], precision=_HI,
                 preferred_element_type=jnp.float32)
    Wn_ref[...] = jnp.concatenate([Wc, cnode], axis=0)
    We_ref[...] = jnp.concatenate([We, cedge], axis=0)


def _node_body(x_ref, b_ref, Wn_ref, o_ref):
    n = x_ref.shape[0]
    f = x_ref.shape[1]
    g = Wn_ref.shape[0] - f
    ohT = (b_ref[...] == lax.broadcasted_iota(jnp.int32, (g, n), 0)
           ).astype(jnp.float32)
    o_ref[...] = (
        jnp.dot(x_ref[...], Wn_ref[0:f, :], precision=_HI,
                preferred_element_type=jnp.float32)
        + lax.dot_general(ohT, Wn_ref[f:, :], (((0,), (0,)), ((), ())),
                          precision=_HI, preferred_element_type=jnp.float32))


def _edge_body(eaT_ref, b_ref, We_ref, o_ref):
    bm = eaT_ref.shape[1]
    g = We_ref.shape[0] - eaT_ref.shape[0]
    i = pl.program_id(0)
    b = b_ref[pl.ds(i * bm, bm)]
    ohT = (b[None, :] == lax.broadcasted_iota(jnp.int32, (g, bm), 0)
           ).astype(jnp.float32)
    xc = jnp.concatenate([eaT_ref[...], ohT], axis=0)
    o_ref[...] = lax.dot_general(We_ref[...], xc, (((0,), (0,)), ((), ())),
                                 precision=_HI,
                                 preferred_element_type=jnp.float32)


def _make_sc_edges(n, e, epw):
    mesh = plsc.VectorSubcoreMesh(core_axis_name="c", subcore_axis_name="s",
                                  num_cores=_NC, num_subcores=_NS)
    unroll = 1
    assert epw % (_LANES * unroll) == 0

    @functools.partial(
        pl.kernel,
        out_type=[
            jax.ShapeDtypeStruct((e,), jnp.float32),
            jax.ShapeDtypeStruct((e,), jnp.float32),
            jax.ShapeDtypeStruct((e * 3,), jnp.float32),
        ],
        mesh=mesh,
        compiler_params=pltpu.CompilerParams(needs_layout_passes=False,
                                             use_tc_tiling_on_sc=False),
        scratch_types=[
            pltpu.VMEM((3 * n,), jnp.float32),
            pltpu.VMEM((epw,), jnp.int32),
            pltpu.VMEM((epw,), jnp.int32),
            pltpu.VMEM((epw,), jnp.float32),
            pltpu.VMEM((epw,), jnp.float32),
            pltpu.VMEM((epw * 3,), jnp.float32),
        ],
    )
    def sc_edges(pos_hbm, eig_hbm, d_hbm, a_hbm, rn_hbm,
                 tab_v, src_v, tgt_v, d_v, a_v, rn_v):
        wid = lax.axis_index("s") * _NC + lax.axis_index("c")
        base = wid * epw
        pltpu.sync_copy(pos_hbm, tab_v)
        pltpu.sync_copy(eig_hbm.at[pl.ds(base, epw)], src_v)
        pltpu.sync_copy(eig_hbm.at[pl.ds(e + base, epw)], tgt_v)

        @plsc.parallel_loop(0, epw, _LANES, unroll=unroll)
        def body(ii):
            if True:
                si = src_v[pl.ds(ii, _LANES)]
                ti = tgt_v[pl.ds(ii, _LANES)]
                sx = plsc.load_gather(tab_v, [si])
                sy = plsc.load_gather(tab_v, [si + n])
                sz = plsc.load_gather(tab_v, [si + 2 * n])
                tx = plsc.load_gather(tab_v, [ti])
                ty = plsc.load_gather(tab_v, [ti + n])
                tz = plsc.load_gather(tab_v, [ti + 2 * n])
                rx = tx - sx
                ry = ty - sy
                rz = tz - sz
                aa = sx * tx + sy * ty + sz * tz
                r2 = jnp.maximum(rx * rx + ry * ry + rz * rz, 1e-6)
                zi = jnp.int32(0x5F3759DF) - (plsc.bitcast(r2, jnp.int32) >> 1)
                z = plsc.bitcast(zi, jnp.float32)
                z = z * (1.5 - 0.5 * r2 * z * z)
                z = z * (1.5 - 0.5 * r2 * z * z)
                dd = r2 * z
                inv = 1.0 / (1.0 + dd)
                d_v[pl.ds(ii, _LANES)] = dd
                a_v[pl.ds(ii, _LANES)] = aa
                rn_v[pl.ds(ii, _LANES)] = rx * inv
                rn_v[pl.ds(epw + ii, _LANES)] = ry * inv
                rn_v[pl.ds(2 * epw + ii, _LANES)] = rz * inv
        pltpu.sync_copy(d_v, d_hbm.at[pl.ds(base, epw)])
        pltpu.sync_copy(a_v, a_hbm.at[pl.ds(base, epw)])
        pltpu.sync_copy(rn_v.at[pl.ds(0, epw)], rn_hbm.at[pl.ds(base, epw)])
        pltpu.sync_copy(rn_v.at[pl.ds(epw, epw)],
                        rn_hbm.at[pl.ds(e + base, epw)])
        pltpu.sync_copy(rn_v.at[pl.ds(2 * epw, epw)],
                        rn_hbm.at[pl.ds(2 * e + base, epw)])

    return sc_edges


def kernel(x, t, pos, edge_index_local, edge_index_global, edge_attr_global,
           batch, batch_edge_global,
           W_ta, b_ta, W_tb, b_tb, W_atom, b_atom, W_at, b_at,
           W_bond, b_bond, W_bt, b_bt):
    n, f = x.shape
    g = t.shape[0]
    e = edge_index_global.shape[1]
    s_dim = W_ta.shape[1]
    ed = W_tb.shape[1]
    nb = W_bond.shape[0]
    vd = 64

    batch_row = batch.reshape(1, n)

    posc, Wn, Wec = pl.pallas_call(
        _prep_body,
        out_shape=[
            jax.ShapeDtypeStruct((3 * n,), jnp.float32),
            jax.ShapeDtypeStruct((f + g, s_dim), jnp.float32),
            jax.ShapeDtypeStruct((nb + g, ed), jnp.float32),
        ],
    )(batch_row, pos.T, t, W_ta, b_ta.reshape(1, s_dim), W_tb,
      b_tb.reshape(1, ed), b_atom.reshape(1, s_dim), W_atom, W_at,
      b_at.reshape(1, s_dim), b_bond.reshape(1, ed), W_bond, W_bt,
      b_bt.reshape(1, ed))

    epw = e // _NW
    sc_edges = _make_sc_edges(n, e, epw)
    d, a, rn_flat = sc_edges(posc, edge_index_global.reshape(2 * e))
    r_norm = rn_flat.reshape(3, e).T

    sc_blk = 128
    s = pl.pallas_call(
        _node_body,
        grid=(s_dim // sc_blk,),
        in_specs=[
            pl.BlockSpec((n, f), lambda i: (0, 0)),
            pl.BlockSpec((1, n), lambda i: (0, 0)),
            pl.BlockSpec((f + g, sc_blk), lambda i: (0, i)),
        ],
        out_specs=pl.BlockSpec((n, sc_blk), lambda i: (0, i)),
        out_shape=jax.ShapeDtypeStruct((n, s_dim), jnp.float32),
    )(x, batch_row, Wn)

    be = 16000
    eaT = pl.pallas_call(
        _edge_body,
        grid=(e // be,),
        in_specs=[
            pl.BlockSpec((nb, be), lambda i: (0, i)),
            pl.BlockSpec((e,), lambda i: (0,)),
            pl.BlockSpec((nb + g, ed), lambda i: (0, 0)),
        ],
        out_specs=pl.BlockSpec((ed, be), lambda i: (0, i)),
        out_shape=jax.ShapeDtypeStruct((ed, e), jnp.float32),
    )(edge_attr_global.T, batch_edge_global, Wec)
    ea = eaT.T

    v = jnp.zeros((n, 3, vd), jnp.float32)
    return (s, v, d, a, r_norm, ea)


# speedup vs baseline: 1.0722x; 1.0722x over previous
import functools

import jax
import jax.numpy as jnp
from jax import lax
from jax.experimental import pallas as pl
from jax.experimental.pallas import tpu as pltpu
from jax.experimental.pallas import tpu_sc as plsc

_HI = lax.Precision.DEFAULT

_NC = 2
_NS = 16
_NW = _NC * _NS
_LANES = 16


def _prep_body(batch_ref, posT_ref, t_ref, W_ta_ref, b_ta_ref, W_tb_ref,
               b_tb_ref, b_atom_ref, W_atom_ref, W_at_ref, b_at_ref,
               b_bond_ref, W_bond_ref, W_bt_ref, b_bt_ref,
               posc_ref, Wn_ref, We_ref):
    n = batch_ref.shape[1]
    g = t_ref.shape[0]
    ohT = (batch_ref[...] == lax.broadcasted_iota(jnp.int32, (g, n), 0)
           ).astype(jnp.float32)
    posT4 = jnp.concatenate(
        [posT_ref[...], jnp.ones((1, n), jnp.float32)], axis=0)
    sums = lax.dot_general(posT4, ohT, (((1,), (1,)), ((), ())),
                           precision=_HI, preferred_element_type=jnp.float32)
    mean = sums / jnp.maximum(sums[3:4, :], 1.0)
    poscT = posT4 - lax.dot_general(mean, ohT, (((1,), (0,)), ((), ())),
                                    precision=_HI,
                                    preferred_element_type=jnp.float32)
    posc_ref[pl.ds(0, n)] = poscT[0, :]
    posc_ref[pl.ds(n, n)] = poscT[1, :]
    posc_ref[pl.ds(2 * n, n)] = poscT[2, :]
    ta = t_ref[...] * W_ta_ref[...] + b_ta_ref[...]
    cnode = jnp.dot(ta + b_atom_ref[...], W_at_ref[...],
                    precision=_HI,
                    preferred_element_type=jnp.float32) + b_at_ref[...]
    tb = t_ref[...] * W_tb_ref[...] + b_tb_ref[...]
    cedge = jnp.dot(tb + b_bond_ref[...], W_bt_ref[...],
                    precision=_HI,
                    preferred_element_type=jnp.float32) + b_bt_ref[...]
    Wc = jnp.dot(W_atom_ref[...], W_at_ref[...], precision=_HI,
                 preferred_element_type=jnp.float32)
    We = jnp.dot(W_bond_ref[...], W_bt_ref[...], precision=_HI,
                 preferred_element_type=jnp.float32)
    Wn_ref[...] = jnp.concatenate([Wc, cnode], axis=0)
    We_ref[...] = jnp.concatenate([We, cedge], axis=0)


def _node_body(x_ref, b_ref, Wn_ref, o_ref):
    n = x_ref.shape[0]
    f = x_ref.shape[1]
    g = Wn_ref.shape[0] - f
    ohT = (b_ref[...] == lax.broadcasted_iota(jnp.int32, (g, n), 0)
           ).astype(jnp.float32)
    o_ref[...] = (
        jnp.dot(x_ref[...], Wn_ref[0:f, :], precision=_HI,
                preferred_element_type=jnp.float32)
        + lax.dot_general(ohT, Wn_ref[f:, :], (((0,), (0,)), ((), ())),
                          precision=_HI, preferred_element_type=jnp.float32))


def _edge_body(eaT_ref, b_ref, We_ref, o_ref):
    bm = eaT_ref.shape[1]
    g = We_ref.shape[0] - eaT_ref.shape[0]
    i = pl.program_id(0)
    b = b_ref[pl.ds(i * bm, bm)]
    ohT = (b[None, :] == lax.broadcasted_iota(jnp.int32, (g, bm), 0)
           ).astype(jnp.float32)
    xc = jnp.concatenate([eaT_ref[...], ohT], axis=0)
    o_ref[...] = lax.dot_general(We_ref[...], xc, (((0,), (0,)), ((), ())),
                                 precision=_HI,
                                 preferred_element_type=jnp.float32)


def _make_sc_edges(n, e, epw):
    mesh = plsc.VectorSubcoreMesh(core_axis_name="c", subcore_axis_name="s",
                                  num_cores=_NC, num_subcores=_NS)
    unroll = 1
    assert epw % (_LANES * unroll) == 0

    @functools.partial(
        pl.kernel,
        out_type=[
            jax.ShapeDtypeStruct((e,), jnp.float32),
            jax.ShapeDtypeStruct((e,), jnp.float32),
            jax.ShapeDtypeStruct((e * 3,), jnp.float32),
        ],
        mesh=mesh,
        compiler_params=pltpu.CompilerParams(needs_layout_passes=False,
                                             use_tc_tiling_on_sc=False),
        scratch_types=[
            pltpu.VMEM((3 * n,), jnp.float32),
            pltpu.VMEM((epw,), jnp.int32),
            pltpu.VMEM((epw,), jnp.int32),
            pltpu.VMEM((epw,), jnp.float32),
            pltpu.VMEM((epw,), jnp.float32),
            pltpu.VMEM((epw * 3,), jnp.float32),
        ],
    )
    def sc_edges(pos_hbm, eig_hbm, d_hbm, a_hbm, rn_hbm,
                 tab_v, src_v, tgt_v, d_v, a_v, rn_v):
        wid = lax.axis_index("s") * _NC + lax.axis_index("c")
        base = wid * epw
        pltpu.sync_copy(pos_hbm, tab_v)
        pltpu.sync_copy(eig_hbm.at[pl.ds(base, epw)], src_v)
        pltpu.sync_copy(eig_hbm.at[pl.ds(e + base, epw)], tgt_v)

        @plsc.parallel_loop(0, epw, _LANES, unroll=unroll)
        def body(ii):
            if True:
                si = src_v[pl.ds(ii, _LANES)]
                ti = tgt_v[pl.ds(ii, _LANES)]
                sx = plsc.load_gather(tab_v, [si])
                sy = plsc.load_gather(tab_v, [si + n])
                sz = plsc.load_gather(tab_v, [si + 2 * n])
                tx = plsc.load_gather(tab_v, [ti])
                ty = plsc.load_gather(tab_v, [ti + n])
                tz = plsc.load_gather(tab_v, [ti + 2 * n])
                rx = tx - sx
                ry = ty - sy
                rz = tz - sz
                aa = sx * tx + sy * ty + sz * tz
                r2 = jnp.maximum(rx * rx + ry * ry + rz * rz, 1e-6)
                zi = jnp.int32(0x5F3759DF) - (plsc.bitcast(r2, jnp.int32) >> 1)
                z = plsc.bitcast(zi, jnp.float32)
                z = z * (1.5 - 0.5 * r2 * z * z)
                z = z * (1.5 - 0.5 * r2 * z * z)
                dd = r2 * z
                inv = 1.0 / (1.0 + dd)
                d_v[pl.ds(ii, _LANES)] = dd
                a_v[pl.ds(ii, _LANES)] = aa
                rn_v[pl.ds(ii, _LANES)] = rx * inv
                rn_v[pl.ds(epw + ii, _LANES)] = ry * inv
                rn_v[pl.ds(2 * epw + ii, _LANES)] = rz * inv
        pltpu.sync_copy(d_v, d_hbm.at[pl.ds(base, epw)])
        pltpu.sync_copy(a_v, a_hbm.at[pl.ds(base, epw)])
        pltpu.sync_copy(rn_v.at[pl.ds(0, epw)], rn_hbm.at[pl.ds(base, epw)])
        pltpu.sync_copy(rn_v.at[pl.ds(epw, epw)],
                        rn_hbm.at[pl.ds(e + base, epw)])
        pltpu.sync_copy(rn_v.at[pl.ds(2 * epw, epw)],
                        rn_hbm.at[pl.ds(2 * e + base, epw)])

    return sc_edges


def kernel(x, t, pos, edge_index_local, edge_index_global, edge_attr_global,
           batch, batch_edge_global,
           W_ta, b_ta, W_tb, b_tb, W_atom, b_atom, W_at, b_at,
           W_bond, b_bond, W_bt, b_bt):
    n, f = x.shape
    g = t.shape[0]
    e = edge_index_global.shape[1]
    s_dim = W_ta.shape[1]
    ed = W_tb.shape[1]
    nb = W_bond.shape[0]
    vd = 64

    batch_row = batch.reshape(1, n)

    posc, Wn, Wec = pl.pallas_call(
        _prep_body,
        out_shape=[
            jax.ShapeDtypeStruct((3 * n,), jnp.float32),
            jax.ShapeDtypeStruct((f + g, s_dim), jnp.float32),
            jax.ShapeDtypeStruct((nb + g, ed), jnp.float32),
        ],
    )(batch_row, pos.T, t, W_ta, b_ta.reshape(1, s_dim), W_tb,
      b_tb.reshape(1, ed), b_atom.reshape(1, s_dim), W_atom, W_at,
      b_at.reshape(1, s_dim), b_bond.reshape(1, ed), W_bond, W_bt,
      b_bt.reshape(1, ed))

    epw = e // _NW
    sc_edges = _make_sc_edges(n, e, epw)
    d, a, rn_flat = sc_edges(posc, edge_index_global.reshape(2 * e))
    r_norm = rn_flat.reshape(3, e).T

    sc_blk = 128
    s = pl.pallas_call(
        _node_body,
        grid=(s_dim // sc_blk,),
        in_specs=[
            pl.BlockSpec((n, f), lambda i: (0, 0)),
            pl.BlockSpec((1, n), lambda i: (0, 0)),
            pl.BlockSpec((f + g, sc_blk), lambda i: (0, i)),
        ],
        out_specs=pl.BlockSpec((n, sc_blk), lambda i: (0, i)),
        out_shape=jax.ShapeDtypeStruct((n, s_dim), jnp.float32),
    )(x, batch_row, Wn)

    be = 32000
    eaT = pl.pallas_call(
        _edge_body,
        grid=(e // be,),
        in_specs=[
            pl.BlockSpec((nb, be), lambda i: (0, i)),
            pl.BlockSpec((e,), lambda i: (0,)),
            pl.BlockSpec((nb + g, ed), lambda i: (0, 0)),
        ],
        out_specs=pl.BlockSpec((ed, be), lambda i: (0, i)),
        out_shape=jax.ShapeDtypeStruct((ed, e), jnp.float32),
    )(edge_attr_global.T, batch_edge_global, Wec)
    ea = eaT.T

    v = jnp.zeros((n, 3, vd), jnp.float32)
    return (s, v, d, a, r_norm, ea)
